# Initial kernel scaffold; baseline (speedup 1.0000x reference)
import jax
import jax.numpy as jnp
from jax.experimental import pallas as pl


def kernel(chem_influence, o_pre, w, threshold, decay, src, dst):
    raise NotImplementedError("write your pallas kernel here")


# speedup vs baseline: 14.5070x; 14.5070x over previous
import dataclasses
import functools

import jax
import jax.numpy as jnp
from jax import lax
from jax.experimental import pallas as pl
from jax.experimental.pallas import tpu as pltpu
from jax.experimental.pallas import tpu_sc as plsc

NC = 2
NS = 16
NW = NC * NS
GROUP = 128
CHUNK_GROUPS = 56


def _sc_scatter_add(o_t, src2d, dst2d, w2d, zeros_nb, groups_per_tile):
    n, b = o_t.shape
    n_per_sub = n // NS
    nchunks = groups_per_tile // CHUNK_GROUPS

    def body(o_hbm, src_hbm, dst_hbm, w_hbm, z_hbm, gj_hbm,
             src_v, dst_v, w_v, rows_v, acc_sh, sem_i, sem_g):
        c = lax.axis_index("c")
        s = lax.axis_index("s")
        wid = c * NS + s

        pltpu.sync_copy(z_hbm.at[pl.ds(s * n_per_sub, n_per_sub)],
                        acc_sh.at[pl.ds(s * n_per_sub, n_per_sub)])
        plsc.subcore_barrier()

        iot = lax.broadcasted_iota(jnp.int32, (16,), 0)
        half = iot >> 3
        colv = iot & 7

        for chunk in range(nchunks):
            row0 = wid * groups_per_tile + chunk * CHUNK_GROUPS
            cp_s = pltpu.async_copy(src_hbm.at[pl.ds(row0, CHUNK_GROUPS)], src_v, sem_i)
            cp_d = pltpu.async_copy(dst_hbm.at[pl.ds(row0, CHUNK_GROUPS)], dst_v, sem_i)
            cp_w = pltpu.async_copy(w_hbm.at[pl.ds(row0, CHUNK_GROUPS)], w_v, sem_i)
            cp_s.wait(); cp_d.wait(); cp_w.wait()

            @pl.loop(0, CHUNK_GROUPS)
            def _(j):
                pltpu.async_copy(o_hbm.at[src_v.at[j]], rows_v, sem_g).wait()

                jj = jnp.full((16,), j, jnp.int32)

                @pl.loop(0, GROUP * b // 16)
                def _(k):
                    ridx = 2 * k + half
                    wv = plsc.load_gather(w_v, [jj, ridx])
                    v = plsc.load_gather(rows_v, [ridx, colv])
                    plsc.store_scatter(rows_v, [ridx, colv], jnp.abs(v) * wv)

                pltpu.sync_copy(rows_v, acc_sh.at[dst_v.at[j]], add=True)

        plsc.subcore_barrier()
        pltpu.sync_copy(acc_sh.at[pl.ds(s * n_per_sub, n_per_sub)],
                        gj_hbm.at[c].at[pl.ds(s * n_per_sub, n_per_sub)])

    mesh = plsc.VectorSubcoreMesh(core_axis_name="c", subcore_axis_name="s",
                                  num_cores=NC, num_subcores=NS)
    cp = pltpu.CompilerParams(use_tc_tiling_on_sc=False)
    if "needs_layout_passes" in pltpu.CompilerParams.__dataclass_fields__:
        cp = dataclasses.replace(cp, needs_layout_passes=False)
    return pl.kernel(
        body,
        compiler_params=cp,
        out_type=jax.ShapeDtypeStruct((NC, n, b), jnp.float32),
        mesh=mesh,
        scratch_types=[
            pltpu.VMEM((CHUNK_GROUPS, GROUP), jnp.int32),
            pltpu.VMEM((CHUNK_GROUPS, GROUP), jnp.int32),
            pltpu.VMEM((CHUNK_GROUPS, GROUP), jnp.float32),
            pltpu.VMEM((GROUP, b), jnp.float32),
            pltpu.VMEM_SHARED((n, b), jnp.float32),
            pltpu.SemaphoreType.DMA,
            pltpu.SemaphoreType.DMA,
        ],
    )(o_t, src2d, dst2d, w2d, zeros_nb)


def _tc_elem_kernel(chem_ref, gj0_ref, gj1_ref, t_ref, d_ref, o_ref, e_ref):
    S = jnp.clip(chem_ref[...] + gj0_ref[...] + gj1_ref[...], -10.0, 10.0)
    T = t_ref[...]
    D = d_ref[...]
    new_o = jnp.maximum(S - T, 0.0)
    gt = S > T
    mask = jnp.logical_and(jnp.logical_not(gt), S == 0.0)
    new_e = jnp.where(gt, new_o, jnp.where(mask, -D, S))
    o_ref[...] = new_o
    e_ref[...] = new_e


def _tc_elementwise(chem, gj0, gj1, threshold, decay):
    b, n = chem.shape
    bn = 2048
    grid = (pl.cdiv(n, bn),)
    row_spec = pl.BlockSpec((b, bn), lambda i: (0, i))
    one_spec = pl.BlockSpec((1, bn), lambda i: (0, i))
    return pl.pallas_call(
        _tc_elem_kernel,
        grid=grid,
        in_specs=[row_spec, row_spec, row_spec, one_spec, one_spec],
        out_specs=[row_spec, row_spec],
        out_shape=[jax.ShapeDtypeStruct((b, n), jnp.float32),
                   jax.ShapeDtypeStruct((b, n), jnp.float32)],
    )(chem, gj0, gj1, threshold.reshape(1, n), decay.reshape(1, n))


def kernel(chem_influence, o_pre, w, threshold, decay, src, dst):
    b, n = o_pre.shape
    e = src.shape[0]

    edges_per_tile = NW * GROUP * CHUNK_GROUPS
    e_pad = ((e + edges_per_tile - 1) // edges_per_tile) * edges_per_tile
    groups_per_tile = e_pad // (NW * GROUP)

    pad = e_pad - e
    src_p = jnp.pad(src, (0, pad)).reshape(e_pad // GROUP, GROUP)
    dst_p = jnp.pad(dst, (0, pad)).reshape(e_pad // GROUP, GROUP)
    w_p = jnp.pad(w, (0, pad)).reshape(e_pad // GROUP, GROUP)

    n_pad = ((n + NS * 8 - 1) // (NS * 8)) * (NS * 8)
    o_t = jnp.pad(o_pre.T, ((0, n_pad - n), (0, 0)))
    zeros_nb = jnp.zeros((n_pad, b), jnp.float32)

    gj_p = _sc_scatter_add(o_t, src_p, dst_p, w_p, zeros_nb, groups_per_tile)
    gj_bt = jnp.transpose(gj_p[:, :n, :], (0, 2, 1))

    new_o, new_e = _tc_elementwise(chem_influence, gj_bt[0], gj_bt[1],
                                   threshold, decay)
    return (new_o, (new_e, new_o))


# speedup vs baseline: 23.6254x; 1.6286x over previous
import dataclasses
import functools

import jax
import jax.numpy as jnp
from jax import lax
from jax.experimental import pallas as pl
from jax.experimental.pallas import tpu as pltpu
from jax.experimental.pallas import tpu_sc as plsc

NC = 2
NS = 16
NW = NC * NS
GROUP = 128
CHUNK_GROUPS = 56


def _sc_scatter_add(o_t, src2d, dst2d, w2d, zeros_nb, groups_per_tile):
    n, b = o_t.shape
    n_per_sub = n // NS
    nchunks = groups_per_tile // CHUNK_GROUPS

    def body(o_hbm, src_hbm, dst_hbm, w_hbm, z_hbm, gj_hbm,
             src_v, dst_v, w_v, r0, r1, r2, r3, acc_sh,
             sem_i, sg0, sg1, sg2, sg3, ss0, ss1, ss2, ss3):
        c = lax.axis_index("c")
        s = lax.axis_index("s")
        wid = c * NS + s
        rows = [r0, r1, r2, r3]
        sgs = [sg0, sg1, sg2, sg3]
        sss = [ss0, ss1, ss2, ss3]

        pltpu.sync_copy(z_hbm.at[pl.ds(s * n_per_sub, n_per_sub)],
                        acc_sh.at[pl.ds(s * n_per_sub, n_per_sub)])
        plsc.subcore_barrier()

        iot = lax.broadcasted_iota(jnp.int32, (16,), 0)
        half = iot >> 3
        colv = iot & 7

        def fire_gather(j, i):
            pltpu.async_copy(o_hbm.at[src_v.at[j]], rows[i], sgs[i])

        def wait_gather(i):
            pltpu.make_async_copy(o_hbm.at[src_v.at[0]], rows[i], sgs[i]).wait()

        def fire_scat(j, i):
            pltpu.async_copy(rows[i], acc_sh.at[dst_v.at[j]], sss[i], add=True)

        def wait_scat(i):
            pltpu.make_async_copy(rows[i], acc_sh.at[dst_v.at[0]], sss[i]).wait()

        def scale(j, i):
            jj = jnp.full((16,), j, jnp.int32)
            buf = rows[i]

            @pl.loop(0, GROUP * b // 32)
            def _(k2):
                for u in range(2):
                    k = 2 * k2 + u
                    ridx = 2 * k + half
                    wv = plsc.load_gather(w_v, [jj, ridx])
                    v = plsc.load_gather(buf, [ridx, colv])
                    plsc.store_scatter(buf, [ridx, colv], jnp.abs(v) * wv)

        for chunk in range(nchunks):
            row0 = wid * groups_per_tile + chunk * CHUNK_GROUPS
            cp_s = pltpu.async_copy(src_hbm.at[pl.ds(row0, CHUNK_GROUPS)], src_v, sem_i)
            cp_d = pltpu.async_copy(dst_hbm.at[pl.ds(row0, CHUNK_GROUPS)], dst_v, sem_i)
            cp_w = pltpu.async_copy(w_hbm.at[pl.ds(row0, CHUNK_GROUPS)], w_v, sem_i)
            cp_s.wait(); cp_d.wait(); cp_w.wait()

            for p in range(3):
                fire_gather(p, p)

            @pl.loop(0, CHUNK_GROUPS, step=4)
            def _(g):
                for i in range(4):
                    j = g + i
                    wait_gather(i)
                    scale(j, i)
                    fire_scat(j, i)
                    im1 = (i - 1) % 4

                    @pl.when(j >= 1)
                    def _():
                        wait_scat(im1)

                    @pl.when(j < CHUNK_GROUPS - 3)
                    def _():
                        fire_gather(j + 3, (i + 3) % 4)

            wait_scat(3)

        plsc.subcore_barrier()
        pltpu.sync_copy(acc_sh.at[pl.ds(s * n_per_sub, n_per_sub)],
                        gj_hbm.at[c].at[pl.ds(s * n_per_sub, n_per_sub)])

    mesh = plsc.VectorSubcoreMesh(core_axis_name="c", subcore_axis_name="s",
                                  num_cores=NC, num_subcores=NS)
    cp = pltpu.CompilerParams(use_tc_tiling_on_sc=False)
    if "needs_layout_passes" in pltpu.CompilerParams.__dataclass_fields__:
        cp = dataclasses.replace(cp, needs_layout_passes=False)
    return pl.kernel(
        body,
        compiler_params=cp,
        out_type=jax.ShapeDtypeStruct((NC, n, b), jnp.float32),
        mesh=mesh,
        scratch_types=[
            pltpu.VMEM((CHUNK_GROUPS, GROUP), jnp.int32),
            pltpu.VMEM((CHUNK_GROUPS, GROUP), jnp.int32),
            pltpu.VMEM((CHUNK_GROUPS, GROUP), jnp.float32),
            pltpu.VMEM((GROUP, b), jnp.float32),
            pltpu.VMEM((GROUP, b), jnp.float32),
            pltpu.VMEM((GROUP, b), jnp.float32),
            pltpu.VMEM((GROUP, b), jnp.float32),
            pltpu.VMEM_SHARED((n, b), jnp.float32),
        ] + [pltpu.SemaphoreType.DMA] * 9,
    )(o_t, src2d, dst2d, w2d, zeros_nb)


def _tc_elem_kernel(chem_ref, gj0_ref, gj1_ref, t_ref, d_ref, o_ref, e_ref):
    S = jnp.clip(chem_ref[...] + gj0_ref[...] + gj1_ref[...], -10.0, 10.0)
    T = t_ref[...]
    D = d_ref[...]
    new_o = jnp.maximum(S - T, 0.0)
    gt = S > T
    mask = jnp.logical_and(jnp.logical_not(gt), S == 0.0)
    new_e = jnp.where(gt, new_o, jnp.where(mask, -D, S))
    o_ref[...] = new_o
    e_ref[...] = new_e


def _tc_elementwise(chem, gj0, gj1, threshold, decay):
    b, n = chem.shape
    bn = 2048
    grid = (pl.cdiv(n, bn),)
    row_spec = pl.BlockSpec((b, bn), lambda i: (0, i))
    one_spec = pl.BlockSpec((1, bn), lambda i: (0, i))
    return pl.pallas_call(
        _tc_elem_kernel,
        grid=grid,
        in_specs=[row_spec, row_spec, row_spec, one_spec, one_spec],
        out_specs=[row_spec, row_spec],
        out_shape=[jax.ShapeDtypeStruct((b, n), jnp.float32),
                   jax.ShapeDtypeStruct((b, n), jnp.float32)],
    )(chem, gj0, gj1, threshold.reshape(1, n), decay.reshape(1, n))


def kernel(chem_influence, o_pre, w, threshold, decay, src, dst):
    b, n = o_pre.shape
    e = src.shape[0]

    edges_per_tile = NW * GROUP * CHUNK_GROUPS
    e_pad = ((e + edges_per_tile - 1) // edges_per_tile) * edges_per_tile
    groups_per_tile = e_pad // (NW * GROUP)

    pad = e_pad - e
    src_p = jnp.pad(src, (0, pad)).reshape(e_pad // GROUP, GROUP)
    dst_p = jnp.pad(dst, (0, pad)).reshape(e_pad // GROUP, GROUP)
    w_p = jnp.pad(w, (0, pad)).reshape(e_pad // GROUP, GROUP)

    n_pad = ((n + NS * 8 - 1) // (NS * 8)) * (NS * 8)
    o_t = jnp.pad(o_pre.T, ((0, n_pad - n), (0, 0)))
    zeros_nb = jnp.zeros((n_pad, b), jnp.float32)

    gj_p = _sc_scatter_add(o_t, src_p, dst_p, w_p, zeros_nb, groups_per_tile)
    gj_bt = jnp.transpose(gj_p[:, :n, :], (0, 2, 1))

    new_o, new_e = _tc_elementwise(chem_influence, gj_bt[0], gj_bt[1],
                                   threshold, decay)
    return (new_o, (new_e, new_o))


# speedup vs baseline: 30.8422x; 1.3055x over previous
import dataclasses
import functools

import jax
import jax.numpy as jnp
from jax import lax
from jax.experimental import pallas as pl
from jax.experimental.pallas import tpu as pltpu
from jax.experimental.pallas import tpu_sc as plsc

NC = 2
NS = 16
NW = NC * NS
GROUP = 128
CHUNK_GROUPS = 56
NBUF = 4


def _sc_scatter_add(o_t, src2d, dst2d, w2d, zeros_nb):
    n, b = o_t.shape
    gtotal = src2d.shape[0]
    b4 = (gtotal // NW) & ~(NBUF - 1)
    ntile_hi = (gtotal - NW * b4) // NBUF
    nfull = b4 // CHUNK_GROUPS
    tail_lo = b4 - nfull * CHUNK_GROUPS
    tail_hi = tail_lo + NBUF
    n_per_sub = n // NS
    vregs_per_group = GROUP * b // 16

    def body(o_hbm, src_hbm, dst_hbm, w_hbm, z_hbm, gj_hbm,
             src_v, dst_v, w_v, r0, r1, r2, r3, acc_sh,
             sem_i, sg0, sg1, sg2, sg3, ss0, ss1, ss2, ss3):
        c = lax.axis_index("c")
        s = lax.axis_index("s")
        wid = c * NS + s
        rows = [r0, r1, r2, r3]
        sgs = [sg0, sg1, sg2, sg3]
        sss = [ss0, ss1, ss2, ss3]

        pltpu.sync_copy(z_hbm.at[pl.ds(s * n_per_sub, n_per_sub)],
                        acc_sh.at[pl.ds(s * n_per_sub, n_per_sub)])
        plsc.subcore_barrier()

        iot = lax.broadcasted_iota(jnp.int32, (16,), 0)
        half = iot >> 3
        colv = iot & 7

        def fire_gather(j, i):
            pltpu.async_copy(o_hbm.at[src_v.at[j]], rows[i], sgs[i])

        def wait_gather(i):
            pltpu.make_async_copy(o_hbm.at[src_v.at[0]], rows[i], sgs[i]).wait()

        def fire_scat(j, i):
            pltpu.async_copy(rows[i], acc_sh.at[dst_v.at[j]], sss[i], add=True)

        def wait_scat(i):
            pltpu.make_async_copy(rows[i], acc_sh.at[dst_v.at[0]], sss[i]).wait()

        def scale(j, i):
            jj = jnp.full((16,), j, jnp.int32)
            buf = rows[i]

            @pl.loop(0, vregs_per_group // 2)
            def _(k2):
                for u in range(2):
                    k = 2 * k2 + u
                    ridx = 2 * k + half
                    wv = plsc.load_gather(w_v, [jj, ridx])
                    v = plsc.load_gather(buf, [ridx, colv])
                    plsc.store_scatter(buf, [ridx, colv], jnp.abs(v) * wv)

        def stage_and_pipe(row0, cg):
            cp_s = pltpu.async_copy(src_hbm.at[pl.ds(row0, cg)],
                                    src_v.at[pl.ds(0, cg)], sem_i)
            cp_d = pltpu.async_copy(dst_hbm.at[pl.ds(row0, cg)],
                                    dst_v.at[pl.ds(0, cg)], sem_i)
            cp_w = pltpu.async_copy(w_hbm.at[pl.ds(row0, cg)],
                                    w_v.at[pl.ds(0, cg)], sem_i)
            cp_s.wait(); cp_d.wait(); cp_w.wait()

            for p in range(NBUF - 1):
                fire_gather(p, p)

            @pl.loop(0, cg, step=NBUF)
            def _(g):
                for i in range(NBUF):
                    j = g + i
                    wait_gather(i)
                    scale(j, i)
                    fire_scat(j, i)

                    @pl.when(j >= 1)
                    def _():
                        wait_scat((i - 1) % NBUF)

                    @pl.when(j < cg - (NBUF - 1))
                    def _():
                        fire_gather(j + NBUF - 1, (i + NBUF - 1) % NBUF)

            wait_scat((cg - 1) % NBUF)

        start = wid * b4 + NBUF * jnp.minimum(wid, ntile_hi)

        if nfull:
            @pl.loop(0, nfull)
            def _(ci):
                stage_and_pipe(start + ci * CHUNK_GROUPS, CHUNK_GROUPS)

        if tail_lo:
            @pl.when(wid >= ntile_hi)
            def _():
                stage_and_pipe(start + nfull * CHUNK_GROUPS, tail_lo)

        if ntile_hi:
            @pl.when(wid < ntile_hi)
            def _():
                stage_and_pipe(start + nfull * CHUNK_GROUPS, tail_hi)

        plsc.subcore_barrier()
        pltpu.sync_copy(acc_sh.at[pl.ds(s * n_per_sub, n_per_sub)],
                        gj_hbm.at[c].at[pl.ds(s * n_per_sub, n_per_sub)])

    mesh = plsc.VectorSubcoreMesh(core_axis_name="c", subcore_axis_name="s",
                                  num_cores=NC, num_subcores=NS)
    cp = pltpu.CompilerParams(use_tc_tiling_on_sc=False)
    if "needs_layout_passes" in pltpu.CompilerParams.__dataclass_fields__:
        cp = dataclasses.replace(cp, needs_layout_passes=False)
    return pl.kernel(
        body,
        compiler_params=cp,
        out_type=jax.ShapeDtypeStruct((NC, n, b), jnp.float32),
        mesh=mesh,
        scratch_types=[
            pltpu.VMEM((CHUNK_GROUPS, GROUP), jnp.int32),
            pltpu.VMEM((CHUNK_GROUPS, GROUP), jnp.int32),
            pltpu.VMEM((CHUNK_GROUPS, GROUP), jnp.float32),
            pltpu.VMEM((GROUP, b), jnp.float32),
            pltpu.VMEM((GROUP, b), jnp.float32),
            pltpu.VMEM((GROUP, b), jnp.float32),
            pltpu.VMEM((GROUP, b), jnp.float32),
            pltpu.VMEM_SHARED((n, b), jnp.float32),
        ] + [pltpu.SemaphoreType.DMA] * 9,
    )(o_t, src2d, dst2d, w2d, zeros_nb)


def _tc_elem_kernel(chem_ref, gj0_ref, gj1_ref, t_ref, d_ref, o_ref, e_ref):
    S = jnp.clip(chem_ref[...] + gj0_ref[...] + gj1_ref[...], -10.0, 10.0)
    T = t_ref[...]
    D = d_ref[...]
    new_o = jnp.maximum(S - T, 0.0)
    gt = S > T
    mask = jnp.logical_and(jnp.logical_not(gt), S == 0.0)
    new_e = jnp.where(gt, new_o, jnp.where(mask, -D, S))
    o_ref[...] = new_o
    e_ref[...] = new_e


def _tc_elementwise(chem, gj0, gj1, threshold, decay):
    b, n = chem.shape
    bn = 2048
    grid = (pl.cdiv(n, bn),)
    row_spec = pl.BlockSpec((b, bn), lambda i: (0, i))
    one_spec = pl.BlockSpec((1, bn), lambda i: (0, i))
    return pl.pallas_call(
        _tc_elem_kernel,
        grid=grid,
        in_specs=[row_spec, row_spec, row_spec, one_spec, one_spec],
        out_specs=[row_spec, row_spec],
        out_shape=[jax.ShapeDtypeStruct((b, n), jnp.float32),
                   jax.ShapeDtypeStruct((b, n), jnp.float32)],
    )(chem, gj0, gj1, threshold.reshape(1, n), decay.reshape(1, n))


def kernel(chem_influence, o_pre, w, threshold, decay, src, dst):
    b, n = o_pre.shape
    e = src.shape[0]

    unit = GROUP * NBUF
    if e % unit:
        pad = unit - e % unit
        src = jnp.pad(src, (0, pad))
        dst = jnp.pad(dst, (0, pad))
        w = jnp.pad(w, (0, pad))
        e += pad
    src2d = src.reshape(e // GROUP, GROUP)
    dst2d = dst.reshape(e // GROUP, GROUP)
    w2d = w.reshape(e // GROUP, GROUP)

    o_t = o_pre.T
    zeros_nb = jnp.zeros((n, b), jnp.float32)

    gj_p = _sc_scatter_add(o_t, src2d, dst2d, w2d, zeros_nb)
    gj_bt = jnp.transpose(gj_p, (0, 2, 1))

    new_o, new_e = _tc_elementwise(chem_influence, gj_bt[0], gj_bt[1],
                                   threshold, decay)
    return (new_o, (new_e, new_o))


# speedup vs baseline: 30.8865x; 1.0014x over previous
import dataclasses
import functools

import jax
import jax.numpy as jnp
from jax import lax
from jax.experimental import pallas as pl
from jax.experimental.pallas import tpu as pltpu
from jax.experimental.pallas import tpu_sc as plsc

NC = 2
NS = 16
NW = NC * NS
GROUP = 128
CHUNK_GROUPS = 56
NBUF = 4


def _sc_scatter_add(o_t, src1d, dst1d, w1d, zeros_nb):
    n, b = o_t.shape
    gtotal = src1d.shape[0] // GROUP
    b4 = (gtotal // NW) & ~(NBUF - 1)
    ntile_hi = (gtotal - NW * b4) // NBUF
    nfull = b4 // CHUNK_GROUPS
    tail_lo = b4 - nfull * CHUNK_GROUPS
    tail_hi = tail_lo + NBUF
    n_per_sub = n // NS
    vregs_per_group = GROUP * b // 16

    def body(o_hbm, src_hbm, dst_hbm, w_hbm, z_hbm, gj_hbm,
             src_v, dst_v, w_v, r0, r1, r2, r3, acc_sh,
             sem_i, sg0, sg1, sg2, sg3, ss0, ss1, ss2, ss3):
        c = lax.axis_index("c")
        s = lax.axis_index("s")
        wid = c * NS + s
        rows = [r0, r1, r2, r3]
        sgs = [sg0, sg1, sg2, sg3]
        sss = [ss0, ss1, ss2, ss3]

        pltpu.sync_copy(z_hbm.at[pl.ds(s * n_per_sub, n_per_sub)],
                        acc_sh.at[pl.ds(s * n_per_sub, n_per_sub)])
        plsc.subcore_barrier()

        iot = lax.broadcasted_iota(jnp.int32, (16,), 0)
        half = iot >> 3
        colv = iot & 7

        def fire_gather(j, i):
            pltpu.async_copy(o_hbm.at[src_v.at[pl.ds(j * GROUP, GROUP)]],
                             rows[i], sgs[i])

        def wait_gather(i):
            pltpu.make_async_copy(o_hbm.at[src_v.at[pl.ds(0, GROUP)]],
                                 rows[i], sgs[i]).wait()

        def fire_scat(j, i):
            pltpu.async_copy(rows[i], acc_sh.at[dst_v.at[pl.ds(j * GROUP, GROUP)]],
                             sss[i], add=True)

        def wait_scat(i):
            pltpu.make_async_copy(rows[i], acc_sh.at[dst_v.at[pl.ds(0, GROUP)]],
                                 sss[i]).wait()

        def scale(j, i):
            wbase = j * GROUP + half
            buf = rows[i]

            @pl.loop(0, vregs_per_group // 2)
            def _(k2):
                for u in range(2):
                    k = 2 * k2 + u
                    ridx = 2 * k + half
                    wv = plsc.load_gather(w_v, [wbase + 2 * k])
                    v = plsc.load_gather(buf, [ridx, colv])
                    plsc.store_scatter(buf, [ridx, colv], jnp.abs(v) * wv)

        def stage_and_pipe(row0, cg):
            el0 = row0 * GROUP
            ne = cg * GROUP
            cp_s = pltpu.async_copy(src_hbm.at[pl.ds(el0, ne)],
                                    src_v.at[pl.ds(0, ne)], sem_i)
            cp_d = pltpu.async_copy(dst_hbm.at[pl.ds(el0, ne)],
                                    dst_v.at[pl.ds(0, ne)], sem_i)
            cp_w = pltpu.async_copy(w_hbm.at[pl.ds(el0, ne)],
                                    w_v.at[pl.ds(0, ne)], sem_i)
            cp_s.wait(); cp_d.wait(); cp_w.wait()

            for p in range(NBUF - 1):
                fire_gather(p, p)

            @pl.loop(0, cg, step=NBUF)
            def _(g):
                for i in range(NBUF):
                    j = g + i
                    wait_gather(i)
                    scale(j, i)
                    fire_scat(j, i)

                    @pl.when(j >= 1)
                    def _():
                        wait_scat((i - 1) % NBUF)

                    @pl.when(j < cg - (NBUF - 1))
                    def _():
                        fire_gather(j + NBUF - 1, (i + NBUF - 1) % NBUF)

            wait_scat((cg - 1) % NBUF)

        start = wid * b4 + NBUF * jnp.minimum(wid, ntile_hi)

        if nfull:
            @pl.loop(0, nfull)
            def _(ci):
                stage_and_pipe(start + ci * CHUNK_GROUPS, CHUNK_GROUPS)

        if tail_lo:
            @pl.when(wid >= ntile_hi)
            def _():
                stage_and_pipe(start + nfull * CHUNK_GROUPS, tail_lo)

        if ntile_hi:
            @pl.when(wid < ntile_hi)
            def _():
                stage_and_pipe(start + nfull * CHUNK_GROUPS, tail_hi)

        plsc.subcore_barrier()
        pltpu.sync_copy(acc_sh.at[pl.ds(s * n_per_sub, n_per_sub)],
                        gj_hbm.at[c].at[pl.ds(s * n_per_sub, n_per_sub)])

    mesh = plsc.VectorSubcoreMesh(core_axis_name="c", subcore_axis_name="s",
                                  num_cores=NC, num_subcores=NS)
    cp = pltpu.CompilerParams(use_tc_tiling_on_sc=False)
    if "needs_layout_passes" in pltpu.CompilerParams.__dataclass_fields__:
        cp = dataclasses.replace(cp, needs_layout_passes=False)
    return pl.kernel(
        body,
        compiler_params=cp,
        out_type=jax.ShapeDtypeStruct((NC, n, b), jnp.float32),
        mesh=mesh,
        scratch_types=[
            pltpu.VMEM((CHUNK_GROUPS * GROUP,), jnp.int32),
            pltpu.VMEM((CHUNK_GROUPS * GROUP,), jnp.int32),
            pltpu.VMEM((CHUNK_GROUPS * GROUP,), jnp.float32),
            pltpu.VMEM((GROUP, b), jnp.float32),
            pltpu.VMEM((GROUP, b), jnp.float32),
            pltpu.VMEM((GROUP, b), jnp.float32),
            pltpu.VMEM((GROUP, b), jnp.float32),
            pltpu.VMEM_SHARED((n, b), jnp.float32),
        ] + [pltpu.SemaphoreType.DMA] * 9,
    )(o_t, src1d, dst1d, w1d, zeros_nb)


def _tc_elem_kernel(chem_ref, gj0_ref, gj1_ref, t_ref, d_ref, o_ref, e_ref):
    S = jnp.clip(chem_ref[...] + gj0_ref[...] + gj1_ref[...], -10.0, 10.0)
    T = t_ref[...]
    D = d_ref[...]
    new_o = jnp.maximum(S - T, 0.0)
    gt = S > T
    mask = jnp.logical_and(jnp.logical_not(gt), S == 0.0)
    new_e = jnp.where(gt, new_o, jnp.where(mask, -D, S))
    o_ref[...] = new_o
    e_ref[...] = new_e


def _tc_elementwise(chem, gj0, gj1, threshold, decay):
    b, n = chem.shape
    bn = 2048
    grid = (pl.cdiv(n, bn),)
    row_spec = pl.BlockSpec((b, bn), lambda i: (0, i))
    one_spec = pl.BlockSpec((1, bn), lambda i: (0, i))
    return pl.pallas_call(
        _tc_elem_kernel,
        grid=grid,
        in_specs=[row_spec, row_spec, row_spec, one_spec, one_spec],
        out_specs=[row_spec, row_spec],
        out_shape=[jax.ShapeDtypeStruct((b, n), jnp.float32),
                   jax.ShapeDtypeStruct((b, n), jnp.float32)],
    )(chem, gj0, gj1, threshold.reshape(1, n), decay.reshape(1, n))


def kernel(chem_influence, o_pre, w, threshold, decay, src, dst):
    b, n = o_pre.shape
    e = src.shape[0]

    unit = GROUP * NBUF
    if e % unit:
        pad = unit - e % unit
        src = jnp.pad(src, (0, pad))
        dst = jnp.pad(dst, (0, pad))
        w = jnp.pad(w, (0, pad))
        e += pad

    o_t = o_pre.T
    zeros_nb = jnp.zeros((n, b), jnp.float32)

    gj_p = _sc_scatter_add(o_t, src, dst, w, zeros_nb)
    gj_bt = jnp.transpose(gj_p, (0, 2, 1))

    new_o, new_e = _tc_elementwise(chem_influence, gj_bt[0], gj_bt[1],
                                   threshold, decay)
    return (new_o, (new_e, new_o))


# speedup vs baseline: 31.7591x; 1.0282x over previous
import dataclasses
import functools

import jax
import jax.numpy as jnp
from jax import lax
from jax.experimental import pallas as pl
from jax.experimental.pallas import tpu as pltpu
from jax.experimental.pallas import tpu_sc as plsc

NC = 2
NS = 16
NW = NC * NS
GROUP = 128
CHUNK_GROUPS = 56
NBUF = 4


def _sc_scatter_add(o_t, src1d, dst1d, w1d, zeros_nb):
    n, b = o_t.shape
    gtotal = src1d.shape[0] // GROUP
    b4 = (gtotal // NW) & ~(NBUF - 1)
    ntile_hi = (gtotal - NW * b4) // NBUF
    nfull = b4 // CHUNK_GROUPS
    tail_lo = b4 - nfull * CHUNK_GROUPS
    tail_hi = tail_lo + NBUF
    n_per_sub = n // NS
    vregs_per_group = GROUP * b // 16

    def body(o_hbm, src_hbm, dst_hbm, w_hbm, z_hbm, gj_hbm,
             src_v, dst_v, w_v, r0, r1, r2, r3, acc_sh,
             sem_i, sg0, sg1, sg2, sg3, ss0, ss1, ss2, ss3):
        c = lax.axis_index("c")
        s = lax.axis_index("s")
        wid = c * NS + s
        rows = [r0, r1, r2, r3]
        sgs = [sg0, sg1, sg2, sg3]
        sss = [ss0, ss1, ss2, ss3]

        pltpu.sync_copy(z_hbm.at[pl.ds(s * n_per_sub, n_per_sub)],
                        acc_sh.at[pl.ds(s * n_per_sub, n_per_sub)])
        plsc.subcore_barrier()

        iot = lax.broadcasted_iota(jnp.int32, (16,), 0)
        half = iot >> 3
        colv = iot & 7

        def fire_gather(j, i):
            pltpu.async_copy(o_hbm.at[src_v.at[pl.ds(j * GROUP, GROUP)]],
                             rows[i], sgs[i])

        def wait_gather(i):
            pltpu.make_async_copy(o_hbm.at[src_v.at[pl.ds(0, GROUP)]],
                                 rows[i], sgs[i]).wait()

        def fire_scat(j, i):
            pltpu.async_copy(rows[i], acc_sh.at[dst_v.at[pl.ds(j * GROUP, GROUP)]],
                             sss[i], add=True)

        def wait_scat(i):
            pltpu.make_async_copy(rows[i], acc_sh.at[dst_v.at[pl.ds(0, GROUP)]],
                                 sss[i]).wait()

        def scale(j, i):
            wbase = j * GROUP + half
            buf = rows[i]

            @pl.loop(0, vregs_per_group // 2)
            def _(k2):
                for u in range(2):
                    k = 2 * k2 + u
                    ridx = 2 * k + half
                    wv = plsc.load_gather(w_v, [wbase + 2 * k])
                    v = plsc.load_gather(buf, [ridx, colv])
                    plsc.store_scatter(buf, [ridx, colv], jnp.abs(v) * wv)

        def stage_and_pipe(row0, cg):
            el0 = row0 * GROUP
            ne = cg * GROUP
            cp_s = pltpu.async_copy(src_hbm.at[pl.ds(el0, ne)],
                                    src_v.at[pl.ds(0, ne)], sem_i)
            cp_d = pltpu.async_copy(dst_hbm.at[pl.ds(el0, ne)],
                                    dst_v.at[pl.ds(0, ne)], sem_i)
            cp_w = pltpu.async_copy(w_hbm.at[pl.ds(el0, ne)],
                                    w_v.at[pl.ds(0, ne)], sem_i)
            cp_s.wait(); cp_d.wait(); cp_w.wait()

            for p in range(NBUF - 1):
                fire_gather(p, p)

            @pl.loop(0, cg, step=NBUF)
            def _(g):
                for i in range(NBUF):
                    j = g + i
                    wait_gather(i)
                    scale(j, i)
                    fire_scat(j, i)

                    @pl.when(j >= 1)
                    def _():
                        wait_scat((i - 1) % NBUF)

                    @pl.when(j < cg - (NBUF - 1))
                    def _():
                        fire_gather(j + NBUF - 1, (i + NBUF - 1) % NBUF)

            wait_scat((cg - 1) % NBUF)

        start = wid * b4 + NBUF * jnp.minimum(wid, ntile_hi)

        if nfull:
            @pl.loop(0, nfull)
            def _(ci):
                stage_and_pipe(start + ci * CHUNK_GROUPS, CHUNK_GROUPS)

        if tail_lo:
            @pl.when(wid >= ntile_hi)
            def _():
                stage_and_pipe(start + nfull * CHUNK_GROUPS, tail_lo)

        if ntile_hi:
            @pl.when(wid < ntile_hi)
            def _():
                stage_and_pipe(start + nfull * CHUNK_GROUPS, tail_hi)

        plsc.subcore_barrier()
        pltpu.sync_copy(acc_sh.at[pl.ds(s * n_per_sub, n_per_sub)],
                        gj_hbm.at[c].at[pl.ds(s * n_per_sub, n_per_sub)])

    mesh = plsc.VectorSubcoreMesh(core_axis_name="c", subcore_axis_name="s",
                                  num_cores=NC, num_subcores=NS)
    cp = pltpu.CompilerParams(use_tc_tiling_on_sc=False)
    if "needs_layout_passes" in pltpu.CompilerParams.__dataclass_fields__:
        cp = dataclasses.replace(cp, needs_layout_passes=False)
    return pl.kernel(
        body,
        compiler_params=cp,
        out_type=jax.ShapeDtypeStruct((NC, n, b), jnp.float32),
        mesh=mesh,
        scratch_types=[
            pltpu.VMEM((CHUNK_GROUPS * GROUP,), jnp.int32),
            pltpu.VMEM((CHUNK_GROUPS * GROUP,), jnp.int32),
            pltpu.VMEM((CHUNK_GROUPS * GROUP,), jnp.float32),
            pltpu.VMEM((GROUP, b), jnp.float32),
            pltpu.VMEM((GROUP, b), jnp.float32),
            pltpu.VMEM((GROUP, b), jnp.float32),
            pltpu.VMEM((GROUP, b), jnp.float32),
            pltpu.VMEM_SHARED((n, b), jnp.float32),
        ] + [pltpu.SemaphoreType.DMA] * 9,
    )(o_t, src1d, dst1d, w1d, zeros_nb)


def _tc_elem_kernel(chem_ref, gjp_ref, t_ref, d_ref, o_ref, e_ref):
    g = gjp_ref[...]
    gj = jnp.transpose(g[0] + g[1], (1, 0))
    S = jnp.clip(chem_ref[...] + gj, -10.0, 10.0)
    T = t_ref[...]
    D = d_ref[...]
    new_o = jnp.maximum(S - T, 0.0)
    gt = S > T
    mask = jnp.logical_and(jnp.logical_not(gt), S == 0.0)
    new_e = jnp.where(gt, new_o, jnp.where(mask, -D, S))
    o_ref[...] = new_o
    e_ref[...] = new_e


def _tc_elementwise(chem, gj_p, threshold, decay):
    b, n = chem.shape
    bn = 2048
    grid = (pl.cdiv(n, bn),)
    row_spec = pl.BlockSpec((b, bn), lambda i: (0, i))
    gjp_spec = pl.BlockSpec((NC, bn, b), lambda i: (0, i, 0))
    one_spec = pl.BlockSpec((1, bn), lambda i: (0, i))
    return pl.pallas_call(
        _tc_elem_kernel,
        grid=grid,
        in_specs=[row_spec, gjp_spec, one_spec, one_spec],
        out_specs=[row_spec, row_spec],
        out_shape=[jax.ShapeDtypeStruct((b, n), jnp.float32),
                   jax.ShapeDtypeStruct((b, n), jnp.float32)],
    )(chem, gj_p, threshold.reshape(1, n), decay.reshape(1, n))


def kernel(chem_influence, o_pre, w, threshold, decay, src, dst):
    b, n = o_pre.shape
    e = src.shape[0]

    unit = GROUP * NBUF
    if e % unit:
        pad = unit - e % unit
        src = jnp.pad(src, (0, pad))
        dst = jnp.pad(dst, (0, pad))
        w = jnp.pad(w, (0, pad))
        e += pad

    o_t = o_pre.T
    zeros_nb = jnp.zeros((n, b), jnp.float32)

    gj_p = _sc_scatter_add(o_t, src, dst, w, zeros_nb)

    new_o, new_e = _tc_elementwise(chem_influence, gj_p, threshold, decay)
    return (new_o, (new_e, new_o))


# speedup vs baseline: 39.2908x; 1.2372x over previous
import dataclasses
import functools

import jax
import jax.numpy as jnp
from jax import lax
from jax.experimental import pallas as pl
from jax.experimental.pallas import tpu as pltpu
from jax.experimental.pallas import tpu_sc as plsc

NC = 2
NS = 16
NW = NC * NS
GROUP = 128
CHUNK_GROUPS = 56
NBUF = 4
TCH = 7


def _sc_scatter_add(o_t, src1d, dst1d, w1d, zeros_nb):
    n, b = o_t.shape
    np_ = zeros_nb.shape[0]
    ntiles = np_ // 128
    tiles_per_sub = ntiles // NS
    assert tiles_per_sub % TCH == 0
    gtotal = src1d.shape[0] // GROUP
    b4 = (gtotal // NW) & ~(NBUF - 1)
    ntile_hi = (gtotal - NW * b4) // NBUF
    nfull = b4 // CHUNK_GROUPS
    tail_lo = b4 - nfull * CHUNK_GROUPS
    tail_hi = tail_lo + NBUF
    n_per_sub = np_ // NS
    vregs_per_group = GROUP * b // 16

    def body(o_hbm, src_hbm, dst_hbm, w_hbm, z_hbm, gj_hbm,
             src_v, dst_v, w_v, r0, r1, r2, r3, tin, tout, acc_sh,
             sem_i, sg0, sg1, sg2, sg3, ss0, ss1, ss2, ss3):
        c = lax.axis_index("c")
        s = lax.axis_index("s")
        wid = c * NS + s
        rows = [r0, r1, r2, r3]
        sgs = [sg0, sg1, sg2, sg3]
        sss = [ss0, ss1, ss2, ss3]

        pltpu.sync_copy(z_hbm.at[pl.ds(s * n_per_sub, n_per_sub)],
                        acc_sh.at[pl.ds(s * n_per_sub, n_per_sub)])
        plsc.subcore_barrier()

        iot = lax.broadcasted_iota(jnp.int32, (16,), 0)
        half = iot >> 3
        colv = iot & 7

        def fire_gather(j, i):
            pltpu.async_copy(o_hbm.at[src_v.at[pl.ds(j * GROUP, GROUP)]],
                             rows[i], sgs[i])

        def wait_gather(i):
            pltpu.make_async_copy(o_hbm.at[src_v.at[pl.ds(0, GROUP)]],
                                 rows[i], sgs[i]).wait()

        def fire_scat(j, i):
            pltpu.async_copy(rows[i], acc_sh.at[dst_v.at[pl.ds(j * GROUP, GROUP)]],
                             sss[i], add=True)

        def wait_scat(i):
            pltpu.make_async_copy(rows[i], acc_sh.at[dst_v.at[pl.ds(0, GROUP)]],
                                 sss[i]).wait()

        def scale(j, i):
            wbase = j * GROUP + half
            buf = rows[i]

            @pl.loop(0, vregs_per_group // 2)
            def _(k2):
                for u in range(2):
                    k = 2 * k2 + u
                    ridx = 2 * k + half
                    wv = plsc.load_gather(w_v, [wbase + 2 * k])
                    v = plsc.load_gather(buf, [ridx, colv])
                    plsc.store_scatter(buf, [ridx, colv], jnp.abs(v) * wv)

        def stage_and_pipe(row0, cg):
            el0 = row0 * GROUP
            ne = cg * GROUP
            cp_s = pltpu.async_copy(src_hbm.at[pl.ds(el0, ne)],
                                    src_v.at[pl.ds(0, ne)], sem_i)
            cp_d = pltpu.async_copy(dst_hbm.at[pl.ds(el0, ne)],
                                    dst_v.at[pl.ds(0, ne)], sem_i)
            cp_w = pltpu.async_copy(w_hbm.at[pl.ds(el0, ne)],
                                    w_v.at[pl.ds(0, ne)], sem_i)
            cp_s.wait(); cp_d.wait(); cp_w.wait()

            for p in range(NBUF - 1):
                fire_gather(p, p)

            @pl.loop(0, cg, step=NBUF)
            def _(g):
                for i in range(NBUF):
                    j = g + i
                    wait_gather(i)
                    scale(j, i)
                    fire_scat(j, i)

                    @pl.when(j >= 1)
                    def _():
                        wait_scat((i - 1) % NBUF)

                    @pl.when(j < cg - (NBUF - 1))
                    def _():
                        fire_gather(j + NBUF - 1, (i + NBUF - 1) % NBUF)

            wait_scat((cg - 1) % NBUF)

        start = wid * b4 + NBUF * jnp.minimum(wid, ntile_hi)

        if nfull:
            @pl.loop(0, nfull)
            def _(ci):
                stage_and_pipe(start + ci * CHUNK_GROUPS, CHUNK_GROUPS)

        if tail_lo:
            @pl.when(wid >= ntile_hi)
            def _():
                stage_and_pipe(start + nfull * CHUNK_GROUPS, tail_lo)

        if ntile_hi:
            @pl.when(wid < ntile_hi)
            def _():
                stage_and_pipe(start + nfull * CHUNK_GROUPS, tail_hi)

        plsc.subcore_barrier()
        t0 = s * tiles_per_sub

        @pl.loop(0, tiles_per_sub // TCH)
        def _(ch):
            tbase = t0 + ch * TCH
            pltpu.sync_copy(acc_sh.at[pl.ds(tbase * 128, TCH * 128)], tin)

            @pl.loop(0, TCH)
            def _(tt):
                rbase = tt * 128
                for r in range(b):
                    rr = jnp.full((16,), r, jnp.int32)
                    for q in range(8):
                        v = plsc.load_gather(tin, [rbase + 16 * q + iot, rr])
                        tout[tt, r, pl.ds(16 * q, 16)] = v

            pltpu.sync_copy(tout, gj_hbm.at[c].at[pl.ds(tbase, TCH)])

    mesh = plsc.VectorSubcoreMesh(core_axis_name="c", subcore_axis_name="s",
                                  num_cores=NC, num_subcores=NS)
    cp = pltpu.CompilerParams(use_tc_tiling_on_sc=False)
    if "needs_layout_passes" in pltpu.CompilerParams.__dataclass_fields__:
        cp = dataclasses.replace(cp, needs_layout_passes=False)
    return pl.kernel(
        body,
        compiler_params=cp,
        out_type=jax.ShapeDtypeStruct((NC, ntiles, b, 128), jnp.float32),
        mesh=mesh,
        scratch_types=[
            pltpu.VMEM((CHUNK_GROUPS * GROUP,), jnp.int32),
            pltpu.VMEM((CHUNK_GROUPS * GROUP,), jnp.int32),
            pltpu.VMEM((CHUNK_GROUPS * GROUP,), jnp.float32),
            pltpu.VMEM((GROUP, b), jnp.float32),
            pltpu.VMEM((GROUP, b), jnp.float32),
            pltpu.VMEM((GROUP, b), jnp.float32),
            pltpu.VMEM((GROUP, b), jnp.float32),
            pltpu.VMEM((TCH * 128, b), jnp.float32),
            pltpu.VMEM((TCH, b, 128), jnp.float32),
            pltpu.VMEM_SHARED((np_, b), jnp.float32),
        ] + [pltpu.SemaphoreType.DMA] * 9,
    )(o_t, src1d, dst1d, w1d, zeros_nb)


def _tc_elem_kernel(chem_ref, gjp_ref, t_ref, d_ref, o_ref, e_ref):
    g = gjp_ref[...]
    gs = g[0] + g[1]
    bt, b, _ = gs.shape
    gj = jnp.transpose(gs, (1, 0, 2)).reshape(b, bt * 128)
    S = jnp.clip(chem_ref[...] + gj, -10.0, 10.0)
    T = t_ref[...]
    D = d_ref[...]
    new_o = jnp.maximum(S - T, 0.0)
    gt = S > T
    mask = jnp.logical_and(jnp.logical_not(gt), S == 0.0)
    new_e = jnp.where(gt, new_o, jnp.where(mask, -D, S))
    o_ref[...] = new_o
    e_ref[...] = new_e


def _tc_elementwise(chem, gj_p, threshold, decay):
    b, n = chem.shape
    bn = 2048
    grid = (pl.cdiv(n, bn),)
    row_spec = pl.BlockSpec((b, bn), lambda i: (0, i))
    gjp_spec = pl.BlockSpec((NC, bn // 128, b, 128), lambda i: (0, i, 0, 0))
    one_spec = pl.BlockSpec((1, bn), lambda i: (0, i))
    return pl.pallas_call(
        _tc_elem_kernel,
        grid=grid,
        in_specs=[row_spec, gjp_spec, one_spec, one_spec],
        out_specs=[row_spec, row_spec],
        out_shape=[jax.ShapeDtypeStruct((b, n), jnp.float32),
                   jax.ShapeDtypeStruct((b, n), jnp.float32)],
    )(chem, gj_p, threshold.reshape(1, n), decay.reshape(1, n))


def kernel(chem_influence, o_pre, w, threshold, decay, src, dst):
    b, n = o_pre.shape
    e = src.shape[0]

    unit = GROUP * NBUF
    if e % unit:
        pad = unit - e % unit
        src = jnp.pad(src, (0, pad))
        dst = jnp.pad(dst, (0, pad))
        w = jnp.pad(w, (0, pad))
        e += pad

    o_t = o_pre.T
    unit2 = NS * 128 * TCH
    np_ = ((n + unit2 - 1) // unit2) * unit2
    zeros_nb = jnp.zeros((np_, b), jnp.float32)

    gj_p = _sc_scatter_add(o_t, src, dst, w, zeros_nb)

    new_o, new_e = _tc_elementwise(chem_influence, gj_p, threshold, decay)
    return (new_o, (new_e, new_o))
